# baseline (speedup 1.0000x reference)
import jax, jax.numpy as jnp

def _mlp(x, W1, b1, W2, b2):
    return jnp.tanh(x @ W1 + b1) @ W2 + b2

def kernel(news_x, users_x, fc_W1, fc_b1, fc_W2, fc_b2, nc_W1, nc_b1, nc_W2, nc_b2, uc_W1, uc_b1, uc_W2, uc_b2, topic_row, topic_col, user_row, user_col):
    N_TOPIC, N_NEWS, N_USERS = 1000, 10000, 50000
    deg_t = jnp.bincount(topic_col, length=N_TOPIC).astype(jnp.float32)
    agr_t = jax.ops.segment_sum(news_x[topic_row], topic_col, num_segments=N_TOPIC)
    agr_t = agr_t / (deg_t[:, None] + 1e-08)
    topic_feats = _mlp(agr_t, fc_W1, fc_b1, fc_W2, fc_b2)
    deg_n = jnp.bincount(topic_row, length=N_NEWS).astype(jnp.float32)
    agr_n = jax.ops.segment_sum(topic_feats[topic_col], topic_row, num_segments=N_NEWS)
    agr_n = agr_n / (deg_n[:, None] + 1e-08)
    news_out = _mlp(jnp.concatenate([news_x, agr_n], axis=1), nc_W1, nc_b1, nc_W2, nc_b2)
    deg_u = jnp.bincount(user_col, length=N_USERS).astype(jnp.float32)
    agr_u = jax.ops.segment_sum(news_out[user_row], user_col, num_segments=N_USERS)
    agr_u = agr_u / (deg_u[:, None] + 1e-08)
    users_out = _mlp(jnp.concatenate([users_x, agr_u], axis=1), uc_W1, uc_b1, uc_W2, uc_b2)
    return (news_out, topic_feats, users_out)


# speedup vs baseline: 1.7370x; 1.7370x over previous
import functools

import jax
import jax.numpy as jnp
from jax import lax
from jax.experimental import pallas as pl
from jax.experimental.pallas import tpu as pltpu
from jax.experimental.pallas import tpu_sc as plsc

N_NEWS, N_TOPIC, N_USERS, E, D = 10000, 1000, 50000, 320000, 128
NC, NS = 2, 16
B = 80
ST_PAD, SN_PAD = 1024, 10240
UCH, NCHK = 12800, 4
CPAD = 12928
UPAD = UCH * NCHK
f32, i32 = jnp.float32, jnp.int32

_MESH = plsc.VectorSubcoreMesh(
    core_axis_name="c", subcore_axis_name="s", num_cores=NC, num_subcores=NS)


@functools.partial(
    pl.kernel,
    out_type=(
        jax.ShapeDtypeStruct((NC, ST_PAD, D), f32),
        jax.ShapeDtypeStruct((NC, ST_PAD, D), f32),
        jax.ShapeDtypeStruct((NC, SN_PAD, D), f32),
    ),
    mesh=_MESH,
    scratch_types=[
        pltpu.VMEM_SHARED((ST_PAD, D), f32),
        pltpu.VMEM_SHARED((ST_PAD, D), f32),
        pltpu.VMEM_SHARED((SN_PAD, D), f32),
        pltpu.VMEM((B,), i32),
        pltpu.VMEM((B,), i32),
        pltpu.VMEM((B, D), f32),
        pltpu.VMEM((B, D), f32),
        pltpu.SemaphoreType.DMA,
    ],
)
def _stage1(news, trow, tcol, z128, ones_in,
            accT_o, degT_o, degN_o,
            accT, degT, degN, gbuf, sbuf, rows, ones, sem):
    c = lax.axis_index("c")
    s = lax.axis_index("s")
    pltpu.sync_copy(z128.at[pl.ds(0, ST_PAD // NS)],
                    accT.at[pl.ds(s * (ST_PAD // NS), ST_PAD // NS)])
    pltpu.sync_copy(z128.at[pl.ds(0, ST_PAD // NS)],
                    degT.at[pl.ds(s * (ST_PAD // NS), ST_PAD // NS)])
    pltpu.sync_copy(z128.at[pl.ds(0, SN_PAD // NS)],
                    degN.at[pl.ds(s * (SN_PAD // NS), SN_PAD // NS)])
    pltpu.sync_copy(ones_in, ones)
    plsc.subcore_barrier()

    epert = E // (NC * NS)
    ebase = (c * NS + s) * epert

    def body(i, carry):
        b = ebase + i * B
        pltpu.sync_copy(trow.at[pl.ds(b, B)], gbuf)
        pltpu.sync_copy(tcol.at[pl.ds(b, B)], sbuf)
        pltpu.async_copy(news.at[gbuf], rows, sem).wait()
        pltpu.sync_copy(rows, accT.at[sbuf], add=True)
        pltpu.sync_copy(ones, degT.at[sbuf], add=True)
        pltpu.sync_copy(ones, degN.at[gbuf], add=True)
        return carry

    lax.fori_loop(0, epert // B, body, 0)
    plsc.subcore_barrier()
    pltpu.sync_copy(accT.at[pl.ds(s * (ST_PAD // NS), ST_PAD // NS)],
                    accT_o.at[c, pl.ds(s * (ST_PAD // NS), ST_PAD // NS)])
    pltpu.sync_copy(degT.at[pl.ds(s * (ST_PAD // NS), ST_PAD // NS)],
                    degT_o.at[c, pl.ds(s * (ST_PAD // NS), ST_PAD // NS)])
    pltpu.sync_copy(degN.at[pl.ds(s * (SN_PAD // NS), SN_PAD // NS)],
                    degN_o.at[c, pl.ds(s * (SN_PAD // NS), SN_PAD // NS)])


@functools.partial(
    pl.kernel,
    out_type=jax.ShapeDtypeStruct((NC, SN_PAD, D), f32),
    mesh=_MESH,
    scratch_types=[
        pltpu.VMEM_SHARED((SN_PAD, D), f32),
        pltpu.VMEM((B,), i32),
        pltpu.VMEM((B,), i32),
        pltpu.VMEM((B, D), f32),
        pltpu.SemaphoreType.DMA,
    ],
)
def _stage2(tfeat, tcol, trow, z128, accN_o, accN, gbuf, sbuf, rows, sem):
    c = lax.axis_index("c")
    s = lax.axis_index("s")
    pltpu.sync_copy(z128.at[pl.ds(0, SN_PAD // NS)],
                    accN.at[pl.ds(s * (SN_PAD // NS), SN_PAD // NS)])
    plsc.subcore_barrier()

    epert = E // (NC * NS)
    ebase = (c * NS + s) * epert

    def body(i, carry):
        b = ebase + i * B
        pltpu.sync_copy(tcol.at[pl.ds(b, B)], gbuf)
        pltpu.sync_copy(trow.at[pl.ds(b, B)], sbuf)
        pltpu.async_copy(tfeat.at[gbuf], rows, sem).wait()
        pltpu.sync_copy(rows, accN.at[sbuf], add=True)
        return carry

    lax.fori_loop(0, epert // B, body, 0)
    plsc.subcore_barrier()
    pltpu.sync_copy(accN.at[pl.ds(s * (SN_PAD // NS), SN_PAD // NS)],
                    accN_o.at[c, pl.ds(s * (SN_PAD // NS), SN_PAD // NS)])


@functools.partial(
    pl.kernel,
    out_type=jax.ShapeDtypeStruct((UPAD, D), f32),
    mesh=_MESH,
    scratch_types=[
        pltpu.VMEM_SHARED((CPAD, D), f32),
        pltpu.VMEM((B,), i32),
        pltpu.VMEM((B,), i32),
        pltpu.VMEM((B,), i32),
        pltpu.VMEM((B, D), f32),
        pltpu.SemaphoreType.DMA,
    ],
)
def _stage3(nout, urow, ucol, z128, accU_o, acc, gbuf, sbuf, sbuf2, rows, sem):
    c = lax.axis_index("c")
    s = lax.axis_index("s")
    trash = UCH + 8 * s
    epert = E // NS
    for chunk in range(NCHK):
        base = chunk * UCH

        @pl.when(c == chunk // 2)
        def _(base=base):
            pltpu.sync_copy(z128, acc.at[pl.ds(s * (CPAD // NS), CPAD // NS)])
            plsc.subcore_barrier()

            def body(i, carry):
                b = s * epert + i * B
                pltpu.sync_copy(urow.at[pl.ds(b, B)], gbuf)
                pltpu.sync_copy(ucol.at[pl.ds(b, B)], sbuf)
                pltpu.async_copy(nout.at[gbuf], rows, sem).wait()

                def mbody(j, carry2):
                    v = sbuf[pl.ds(j * 16, 16)]
                    lo = v - base
                    ok = (lo >= 0) & (lo < UCH)
                    sbuf2[pl.ds(j * 16, 16)] = jnp.where(ok, lo, trash)
                    return carry2

                lax.fori_loop(0, B // 16, mbody, 0)
                pltpu.sync_copy(rows, acc.at[sbuf2], add=True)
                return carry

            lax.fori_loop(0, epert // B, body, 0)
            plsc.subcore_barrier()
            pltpu.sync_copy(acc.at[pl.ds(s * (UCH // NS), UCH // NS)],
                            accU_o.at[pl.ds(base + s * (UCH // NS),
                                            UCH // NS)])
            plsc.subcore_barrier()


@functools.partial(
    pl.kernel,
    out_type=jax.ShapeDtypeStruct((UPAD, D), f32),
    mesh=_MESH,
    scratch_types=[
        pltpu.VMEM_SHARED((CPAD, D), f32),
        pltpu.VMEM((B,), i32),
        pltpu.VMEM((B,), i32),
        pltpu.VMEM((B, D), f32),
    ],
)
def _deg3(ucol, z128, ones_in, degU_o, deg, sbuf, sbuf2, ones):
    c = lax.axis_index("c")
    s = lax.axis_index("s")
    trash = UCH + 8 * s
    pltpu.sync_copy(ones_in, ones)
    epert = E // NS
    for chunk in range(NCHK):
        base = chunk * UCH

        @pl.when(c == chunk // 2)
        def _(base=base):
            pltpu.sync_copy(z128, deg.at[pl.ds(s * (CPAD // NS), CPAD // NS)])
            plsc.subcore_barrier()

            def body(i, carry):
                b = s * epert + i * B
                pltpu.sync_copy(ucol.at[pl.ds(b, B)], sbuf)

                def mbody(j, carry2):
                    v = sbuf[pl.ds(j * 16, 16)]
                    lo = v - base
                    ok = (lo >= 0) & (lo < UCH)
                    sbuf2[pl.ds(j * 16, 16)] = jnp.where(ok, lo, trash)
                    return carry2

                lax.fori_loop(0, B // 16, mbody, 0)
                pltpu.sync_copy(ones, deg.at[sbuf2], add=True)
                return carry

            lax.fori_loop(0, epert // B, body, 0)
            plsc.subcore_barrier()
            pltpu.sync_copy(deg.at[pl.ds(s * (UCH // NS), UCH // NS)],
                            degU_o.at[pl.ds(base + s * (UCH // NS),
                                            UCH // NS)])
            plsc.subcore_barrier()


def _mlp1_body(accp, degp, W1, b1, W2, b2, out):
    deg = degp[0, :, 0:1] + degp[1, :, 0:1]
    x = (accp[0] + accp[1]) / (deg + 1e-8)
    h = jnp.tanh(jnp.dot(x, W1[...], preferred_element_type=f32) + b1[...])
    out[...] = jnp.dot(h, W2[...], preferred_element_type=f32) + b2[...]


def _mlp2_body(x, accp, degp, W1, b1, W2, b2, out):
    deg = degp[0, :, 0:1] + degp[1, :, 0:1]
    agr = (accp[0] + accp[1]) / (deg + 1e-8)
    xin = jnp.concatenate([x[...], agr], axis=1)
    h = jnp.tanh(jnp.dot(xin, W1[...], preferred_element_type=f32) + b1[...])
    out[...] = jnp.dot(h, W2[...], preferred_element_type=f32) + b2[...]


def _mlp3_body(x, acc, deg16, W1, b1, W2, b2, out):
    deg = deg16[:, 0:1]
    agr = acc[...] / (deg + 1e-8)
    xin = jnp.concatenate([x[...], agr], axis=1)
    h = jnp.tanh(jnp.dot(xin, W1[...], preferred_element_type=f32) + b1[...])
    out[...] = jnp.dot(h, W2[...], preferred_element_type=f32) + b2[...]


def kernel(news_x, users_x, fc_W1, fc_b1, fc_W2, fc_b2,
           nc_W1, nc_b1, nc_W2, nc_b2, uc_W1, uc_b1, uc_W2, uc_b2,
           topic_row, topic_col, user_row, user_col):
    z128 = jnp.zeros((CPAD // NS, D), f32)
    ones128 = jnp.ones((B, D), f32)
    topic_row = topic_row.astype(i32)
    topic_col = topic_col.astype(i32)
    user_row = user_row.astype(i32)
    user_col = user_col.astype(i32)

    accT_p, degT_p, degN_p = _stage1(
        news_x, topic_row, topic_col, z128, ones128)

    tfeat_pad = pl.pallas_call(
        _mlp1_body,
        out_shape=jax.ShapeDtypeStruct((ST_PAD, D), f32),
    )(accT_p, degT_p, fc_W1, fc_b1, fc_W2, fc_b2)

    accN_p = _stage2(tfeat_pad, topic_col, topic_row, z128)

    RB2 = 1000
    news_out = pl.pallas_call(
        _mlp2_body,
        grid=(N_NEWS // RB2,),
        in_specs=[
            pl.BlockSpec((RB2, D), lambda i: (i, 0)),
            pl.BlockSpec((NC, RB2, D), lambda i: (0, i, 0)),
            pl.BlockSpec((NC, RB2, D), lambda i: (0, i, 0)),
            pl.BlockSpec((256, 512), lambda i: (0, 0)),
            pl.BlockSpec((512,), lambda i: (0,)),
            pl.BlockSpec((512, D), lambda i: (0, 0)),
            pl.BlockSpec((D,), lambda i: (0,)),
        ],
        out_specs=pl.BlockSpec((RB2, D), lambda i: (i, 0)),
        out_shape=jax.ShapeDtypeStruct((N_NEWS, D), f32),
    )(news_x, accN_p[:, :N_NEWS], degN_p[:, :N_NEWS],
      nc_W1, nc_b1, nc_W2, nc_b2)

    accU = _stage3(news_out, user_row, user_col, z128)
    degU = _deg3(user_col, z128, ones128)

    RB3 = 2000
    users_out = pl.pallas_call(
        _mlp3_body,
        grid=(N_USERS // RB3,),
        in_specs=[
            pl.BlockSpec((RB3, D), lambda i: (i, 0)),
            pl.BlockSpec((RB3, D), lambda i: (i, 0)),
            pl.BlockSpec((RB3, D), lambda i: (i, 0)),
            pl.BlockSpec((256, 256), lambda i: (0, 0)),
            pl.BlockSpec((256,), lambda i: (0,)),
            pl.BlockSpec((256, D), lambda i: (0, 0)),
            pl.BlockSpec((D,), lambda i: (0,)),
        ],
        out_specs=pl.BlockSpec((RB3, D), lambda i: (i, 0)),
        out_shape=jax.ShapeDtypeStruct((N_USERS, D), f32),
    )(users_x, accU, degU, uc_W1, uc_b1, uc_W2, uc_b2)

    return (news_out, tfeat_pad[:N_TOPIC], users_out)
